# baseline (speedup 1.0000x reference)
import functools
import heapq
from collections import defaultdict

import numpy as np
import jax
import jax.numpy as jnp
from jax import lax
from jax.experimental import pallas as pl
from jax.experimental.pallas import tpu as pltpu
from jax.experimental.pallas import tpu_sc as plsc

_VOCAB = 256
_DIM = 64
_LANES = 16


def _huffman_meta():
    heap = [[w, [n]] for n, w in {i: 1 for i in range(_VOCAB)}.items()]
    heapq.heapify(heap)
    tree = defaultdict(list)
    while len(heap) > 1:
        lo = heapq.heappop(heap)
        hi = heapq.heappop(heap)
        for node in lo[1]:
            tree[node].append((len(heap), 0))
        for node in hi[1]:
            tree[node].append((len(heap), 1))
        heapq.heappush(heap, [lo[0] + hi[0], lo[1] + hi[1]])

    meta = np.zeros((_VOCAB, 128), dtype=np.int32)
    for node in range(_VOCAB):
        path = tree[node]
        for j in range(_LANES):
            if j < len(path):
                parent, direction = path[j]
                meta[node, 0 + j] = parent
                meta[node, 16 + j] = 1 if direction == 1 else -1
            else:
                meta[node, 32 + j] = 30
    return meta


_META_TAB = _huffman_meta()


def _sc_gather_body(tgt16_hbm, meta_hbm, w_hbm, b_hbm,
                    rows_out, aux_out,
                    tgt_v, meta_v, bv_v, aux_v, sem, sem2, sem3):
    cid = lax.axis_index("c")
    sid = lax.axis_index("s")

    @pl.when(jnp.logical_and(cid == 0, sid == 0))
    def _():
        pltpu.sync_copy(tgt16_hbm, tgt_v.at[pl.ds(0, 1)])
        t = tgt_v[...][0]
        pltpu.async_copy(meta_hbm.at[pl.ds(t, 1)], meta_v, sem3).wait()
        pv = meta_v[0, 0:16]
        cp_b = pltpu.async_copy(b_hbm.at[pv], bv_v, sem2)
        row_cps = []
        for j in range(_LANES):
            pj = pv[j]
            row_cps.append(pltpu.async_copy(
                w_hbm.at[pl.ds(pj, 1)], rows_out.at[pl.ds(j, 1)], sem))
        af = meta_v[0, 16:32].astype(jnp.float32)
        cf = meta_v[0, 32:48].astype(jnp.float32)
        cp_b.wait()
        aux_v[0, :] = af
        aux_v[1, :] = af * bv_v[...] + cf
        pltpu.sync_copy(aux_v, aux_out)
        for cp in row_cps:
            cp.wait()


def _sc_gather(tgt, meta, w, b):
    run = functools.partial(
        pl.kernel,
        out_type=[
            jax.ShapeDtypeStruct((_LANES, _DIM), jnp.float32),
            jax.ShapeDtypeStruct((2, _LANES), jnp.float32),
        ],
        mesh=plsc.VectorSubcoreMesh(core_axis_name="c", subcore_axis_name="s",
                                    num_cores=1),
        scratch_types=[
            pltpu.VMEM((_LANES,), jnp.int32),
            pltpu.VMEM((1, 128), jnp.int32),
            pltpu.VMEM((_LANES,), jnp.float32),
            pltpu.VMEM((2, _LANES), jnp.float32),
            pltpu.SemaphoreType.DMA,
            pltpu.SemaphoreType.DMA,
            pltpu.SemaphoreType.DMA,
        ],
    )(_sc_gather_body)
    return run(tgt, meta, w, b)


_TC_BLK = 16384


def _tc_dense_body(emb_ref, rows_ref, aux_ref, out_ref):
    scores = lax.dot_general(rows_ref[...], emb_ref[...],
                             (((1,), (1,)), ((), ())),
                             preferred_element_type=jnp.float32)
    aux_t = aux_ref[...].T
    f = jax.nn.sigmoid(aux_t[:, 0:1] * scores + aux_t[:, 1:2])
    f = f[0:8, :] * f[8:16, :]
    f = f[0:4, :] * f[4:8, :]
    f = f[0:2, :] * f[2:4, :]
    f = f[0:1, :] * f[1:2, :]
    out_ref[...] = f.reshape(out_ref.shape)


def _tc_dense(emb, rows, aux):
    batch, dim = emb.shape
    out = pl.pallas_call(
        _tc_dense_body,
        grid=(batch // _TC_BLK,),
        in_specs=[
            pl.BlockSpec((_TC_BLK, dim), lambda i: (i, 0)),
            pl.BlockSpec((_LANES, dim), lambda i: (0, 0)),
            pl.BlockSpec((2, _LANES), lambda i: (0, 0)),
        ],
        out_specs=pl.BlockSpec((_TC_BLK // 128, 128), lambda i: (i, 0)),
        out_shape=jax.ShapeDtypeStruct((batch // 128, 128), jnp.float32),
    )(emb, rows, aux)
    return out.reshape(batch)


@jax.jit
def kernel(embeddings, target, W, b):
    meta = jnp.asarray(_META_TAB)
    rows, aux = _sc_gather(target.astype(jnp.int32), meta, W, b)
    return _tc_dense(embeddings, rows, aux)


# speedup vs baseline: 1.0208x; 1.0208x over previous
import functools
import heapq
from collections import defaultdict

import numpy as np
import jax
import jax.numpy as jnp
from jax import lax
from jax.experimental import pallas as pl
from jax.experimental.pallas import tpu as pltpu
from jax.experimental.pallas import tpu_sc as plsc

_VOCAB = 256
_DIM = 64
_LANES = 16


def _huffman_meta():
    heap = [[w, [n]] for n, w in {i: 1 for i in range(_VOCAB)}.items()]
    heapq.heapify(heap)
    tree = defaultdict(list)
    while len(heap) > 1:
        lo = heapq.heappop(heap)
        hi = heapq.heappop(heap)
        for node in lo[1]:
            tree[node].append((len(heap), 0))
        for node in hi[1]:
            tree[node].append((len(heap), 1))
        heapq.heappush(heap, [lo[0] + hi[0], lo[1] + hi[1]])

    meta = np.zeros((_VOCAB, 128), dtype=np.int32)
    for node in range(_VOCAB):
        path = tree[node]
        for j in range(_LANES):
            if j < len(path):
                parent, direction = path[j]
                meta[node, 0 + j] = parent
                meta[node, 16 + j] = 1 if direction == 1 else -1
            else:
                meta[node, 32 + j] = 30
    return meta


_META_TAB = _huffman_meta()


def _sc_gather_body(tgt16_hbm, meta_hbm, w_hbm, b_hbm,
                    rows_out, aux_out,
                    tgt_v, meta_v, bv_v, aux_v, sem, sem2, sem3):
    cid = lax.axis_index("c")
    sid = lax.axis_index("s")

    @pl.when(jnp.logical_and(cid == 0, sid == 0))
    def _():
        pltpu.sync_copy(tgt16_hbm, tgt_v.at[pl.ds(0, 1)])
        t = tgt_v[...][0]
        pltpu.async_copy(meta_hbm.at[pl.ds(t, 1)], meta_v, sem3).wait()
        pv = meta_v[0, 0:16]
        cp_b = pltpu.async_copy(b_hbm.at[pv], bv_v, sem2)
        row_cps = []
        for j in range(_LANES):
            pj = pv[j]
            row_cps.append(pltpu.async_copy(
                w_hbm.at[pl.ds(pj, 1)], rows_out.at[pl.ds(j, 1)], sem))
        af = meta_v[0, 16:32].astype(jnp.float32)
        cf = meta_v[0, 32:48].astype(jnp.float32)
        cp_b.wait()
        aux_v[0, :] = af
        aux_v[1, :] = af * bv_v[...] + cf
        pltpu.sync_copy(aux_v, aux_out)
        for cp in row_cps:
            cp.wait()


def _sc_gather(tgt, meta, w, b):
    run = functools.partial(
        pl.kernel,
        out_type=[
            jax.ShapeDtypeStruct((_LANES, _DIM), jnp.float32),
            jax.ShapeDtypeStruct((2, _LANES), jnp.float32),
        ],
        mesh=plsc.VectorSubcoreMesh(core_axis_name="c", subcore_axis_name="s",
                                    num_cores=1),
        scratch_types=[
            pltpu.VMEM((_LANES,), jnp.int32),
            pltpu.VMEM((1, 128), jnp.int32),
            pltpu.VMEM((_LANES,), jnp.float32),
            pltpu.VMEM((2, _LANES), jnp.float32),
            pltpu.SemaphoreType.DMA,
            pltpu.SemaphoreType.DMA,
            pltpu.SemaphoreType.DMA,
        ],
    )(_sc_gather_body)
    return run(tgt, meta, w, b)


_TC_BLK = 8192


def _tc_dense_body(emb_ref, rows_ref, aux_ref, out_ref):
    scores = lax.dot_general(rows_ref[...], emb_ref[...],
                             (((1,), (1,)), ((), ())),
                             preferred_element_type=jnp.float32)
    aux_t = aux_ref[...].T
    f = jax.nn.sigmoid(aux_t[:, 0:1] * scores + aux_t[:, 1:2])
    f = f[0:8, :] * f[8:16, :]
    f = f[0:4, :] * f[4:8, :]
    f = f[0:2, :] * f[2:4, :]
    f = f[0:1, :] * f[1:2, :]
    out_ref[...] = f.reshape(out_ref.shape)


def _tc_dense(emb, rows, aux):
    batch, dim = emb.shape
    out = pl.pallas_call(
        _tc_dense_body,
        grid=(batch // _TC_BLK,),
        in_specs=[
            pl.BlockSpec((_TC_BLK, dim), lambda i: (i, 0)),
            pl.BlockSpec((_LANES, dim), lambda i: (0, 0)),
            pl.BlockSpec((2, _LANES), lambda i: (0, 0)),
        ],
        out_specs=pl.BlockSpec((_TC_BLK // 128, 128), lambda i: (i, 0)),
        out_shape=jax.ShapeDtypeStruct((batch // 128, 128), jnp.float32),
    )(emb, rows, aux)
    return out.reshape(batch)


@jax.jit
def kernel(embeddings, target, W, b):
    meta = jnp.asarray(_META_TAB)
    rows, aux = _sc_gather(target.astype(jnp.int32), meta, W, b)
    return _tc_dense(embeddings, rows, aux)


# speedup vs baseline: 1.0854x; 1.0632x over previous
import functools
import heapq
from collections import defaultdict

import numpy as np
import jax
import jax.numpy as jnp
from jax import lax
from jax.experimental import pallas as pl
from jax.experimental.pallas import tpu as pltpu
from jax.experimental.pallas import tpu_sc as plsc

_VOCAB = 256
_DIM = 64
_LANES = 16


def _huffman_meta():
    heap = [[w, [n]] for n, w in {i: 1 for i in range(_VOCAB)}.items()]
    heapq.heapify(heap)
    tree = defaultdict(list)
    while len(heap) > 1:
        lo = heapq.heappop(heap)
        hi = heapq.heappop(heap)
        for node in lo[1]:
            tree[node].append((len(heap), 0))
        for node in hi[1]:
            tree[node].append((len(heap), 1))
        heapq.heappush(heap, [lo[0] + hi[0], lo[1] + hi[1]])

    meta = np.zeros((_VOCAB, 128), dtype=np.int32)
    for node in range(_VOCAB):
        path = tree[node]
        for j in range(_LANES):
            if j < len(path):
                parent, direction = path[j]
                meta[node, 0 + j] = parent
                meta[node, 16 + j] = 1 if direction == 1 else -1
            else:
                meta[node, 32 + j] = 30
    return meta


_META_TAB = _huffman_meta()


def _sc_gather_body(tgt_hbm, w_hbm, b_hbm,
                    rows_out, aux_out,
                    tgt_v, bv_v, aux_v, sem, sem2):
    cid = lax.axis_index("c")
    sid = lax.axis_index("s")

    @pl.when(jnp.logical_and(cid == 0, sid == 0))
    def _():
        pltpu.sync_copy(tgt_hbm, tgt_v.at[pl.ds(0, 1)])
        t = tgt_v[...][0]
        k = lax.iota(jnp.int32, _LANES)
        tb = jnp.full((_LANES,), t, jnp.int32)
        par = (jnp.int32(_VOCAB) >> k) - 2 - (tb >> (k + 1))
        valid = k < 8
        pv = jnp.where(valid, par, 0)
        sgn = (2 * ((tb >> k) & 1) - 1).astype(jnp.float32)
        a = jnp.where(valid, sgn, jnp.float32(0.0))
        c = jnp.where(valid, jnp.float32(0.0), jnp.float32(30.0))
        cp_b = pltpu.async_copy(b_hbm.at[pv], bv_v, sem2)
        row_cps = []
        for j in range(_LANES):
            pj = pv[j]
            row_cps.append(pltpu.async_copy(
                w_hbm.at[pl.ds(pj, 1)], rows_out.at[pl.ds(j, 1)], sem))
        cp_b.wait()
        aux_v[0, :] = a
        aux_v[1, :] = a * bv_v[...] + c
        pltpu.sync_copy(aux_v, aux_out)
        for cp in row_cps:
            cp.wait()


def _sc_gather(tgt, w, b):
    run = functools.partial(
        pl.kernel,
        out_type=[
            jax.ShapeDtypeStruct((_LANES, _DIM), jnp.float32),
            jax.ShapeDtypeStruct((2, _LANES), jnp.float32),
        ],
        mesh=plsc.VectorSubcoreMesh(core_axis_name="c", subcore_axis_name="s",
                                    num_cores=1),
        scratch_types=[
            pltpu.VMEM((_LANES,), jnp.int32),
            pltpu.VMEM((_LANES,), jnp.float32),
            pltpu.VMEM((2, _LANES), jnp.float32),
            pltpu.SemaphoreType.DMA,
            pltpu.SemaphoreType.DMA,
        ],
    )(_sc_gather_body)
    return run(tgt, w, b)


_TC_BLK = 8192


def _tc_dense_body(emb_ref, rows_ref, aux_ref, out_ref):
    scores = lax.dot_general(rows_ref[...], emb_ref[...],
                             (((1,), (1,)), ((), ())),
                             preferred_element_type=jnp.float32)
    aux_t = aux_ref[...].T
    f = jax.nn.sigmoid(aux_t[:, 0:1] * scores + aux_t[:, 1:2])
    f = f[0:8, :] * f[8:16, :]
    f = f[0:4, :] * f[4:8, :]
    f = f[0:2, :] * f[2:4, :]
    f = f[0:1, :] * f[1:2, :]
    out_ref[...] = f.reshape(out_ref.shape)


def _tc_dense(emb, rows, aux):
    batch, dim = emb.shape
    out = pl.pallas_call(
        _tc_dense_body,
        grid=(batch // _TC_BLK,),
        in_specs=[
            pl.BlockSpec((_TC_BLK, dim), lambda i: (i, 0)),
            pl.BlockSpec((_LANES, dim), lambda i: (0, 0)),
            pl.BlockSpec((2, _LANES), lambda i: (0, 0)),
        ],
        out_specs=pl.BlockSpec((_TC_BLK // 128, 128), lambda i: (i, 0)),
        out_shape=jax.ShapeDtypeStruct((batch // 128, 128), jnp.float32),
    )(emb, rows, aux)
    return out.reshape(batch)


@jax.jit
def kernel(embeddings, target, W, b):
    rows, aux = _sc_gather(target.astype(jnp.int32), W, b)
    return _tc_dense(embeddings, rows, aux)
